# Initial kernel scaffold; baseline (speedup 1.0000x reference)
import jax
import jax.numpy as jnp
from jax.experimental import pallas as pl


def kernel(user_f_list, url_f_list, user_table, url_table, W, b):
    raise NotImplementedError("write your pallas kernel here")


# speedup vs baseline: 1.6622x; 1.6622x over previous
import functools

import jax
import jax.numpy as jnp
from jax import lax
from jax.experimental import pallas as pl
from jax.experimental.pallas import tpu as pltpu
from jax.experimental.pallas import tpu_sc as plsc

F_DIM = 64
ROW_BLK = 512


def _proj_body(t_ref, w_ref, b_ref, o_ref):
    w = w_ref[...]
    wd = w[:, 0] - w[:, 1]
    db = b_ref[0] - b_ref[1]
    o_ref[...] = jnp.sum(t_ref[...] * wd[None, :], axis=1) + db


def _project(table, w_half, b2):
    v = table.shape[0]
    nblk = (v + ROW_BLK - 1) // ROW_BLK
    vp = nblk * ROW_BLK
    return pl.pallas_call(
        _proj_body,
        grid=(nblk,),
        in_specs=[
            pl.BlockSpec((ROW_BLK, F_DIM), lambda i: (i, 0)),
            pl.BlockSpec((F_DIM, 2), lambda i: (0, 0)),
            pl.BlockSpec(memory_space=pltpu.SMEM),
        ],
        out_specs=pl.BlockSpec((ROW_BLK,), lambda i: (i,)),
        out_shape=jax.ShapeDtypeStruct((vp,), jnp.float32),
    )(table, w_half, b2)


def _make_sc_gather(v1p, v2p, batch):
    nw = 32
    bpw = batch // nw
    mesh = plsc.VectorSubcoreMesh(core_axis_name="c", subcore_axis_name="s")

    @functools.partial(
        pl.kernel,
        mesh=mesh,
        compiler_params=pltpu.CompilerParams(needs_layout_passes=False),
        out_type=jax.ShapeDtypeStruct((2 * batch,), jnp.float32),
        scratch_types=[
            pltpu.VMEM((v1p,), jnp.float32),
            pltpu.VMEM((v2p,), jnp.float32),
            pltpu.VMEM((bpw,), jnp.int32),
            pltpu.VMEM((bpw,), jnp.int32),
            pltpu.VMEM((2 * bpw,), jnp.float32),
        ],
    )
    def sc_gather(pu_hbm, pr_hbm, iu_hbm, ir_hbm, out_hbm,
                  pu_v, pr_v, iu_v, ir_v, out_v):
        wid = lax.axis_index("s") * 2 + lax.axis_index("c")
        base = wid * bpw
        pltpu.sync_copy(pu_hbm, pu_v)
        pltpu.sync_copy(pr_hbm, pr_v)
        pltpu.sync_copy(iu_hbm.at[pl.ds(base, bpw)], iu_v)
        pltpu.sync_copy(ir_hbm.at[pl.ds(base, bpw)], ir_v)
        lanes = lax.iota(jnp.int32, 16)
        for i in range(bpw // 16):
            idxu = iu_v[pl.ds(i * 16, 16)]
            idxr = ir_v[pl.ds(i * 16, 16)]
            u = plsc.load_gather(pu_v, [idxu])
            r = plsc.load_gather(pr_v, [idxr])
            d = u + r
            p0 = 1.0 / (1.0 + jnp.exp(-d))
            p1 = 1.0 / (1.0 + jnp.exp(d))
            pos = i * 32 + 2 * lanes
            plsc.store_scatter(out_v, [pos], p0)
            plsc.store_scatter(out_v, [pos + 1], p1)
        pltpu.sync_copy(out_v, out_hbm.at[pl.ds(2 * base, 2 * bpw)])

    return sc_gather


def kernel(user_f_list, url_f_list, user_table, url_table, W, b):
    batch = user_f_list.shape[0]
    iu = user_f_list[:, 0]
    ir = url_f_list[:, 0]
    pu = _project(user_table, W[:F_DIM], b)
    pr = _project(url_table, W[F_DIM:], jnp.zeros_like(b))
    flat = _make_sc_gather(pu.shape[0], pr.shape[0], batch)(pu, pr, iu, ir)
    return flat.reshape(batch, 2)
